# Initial kernel scaffold; baseline (speedup 1.0000x reference)
import jax
import jax.numpy as jnp
from jax.experimental import pallas as pl


def kernel(x, edge_index, batch, W1a, b1a, W1b, b1b, W2, b2, W3, b3, Wf1, bf1, Wf2, bf2):
    raise NotImplementedError("write your pallas kernel here")


# speedup vs baseline: 2.6486x; 2.6486x over previous
import functools

import jax
import jax.numpy as jnp
from jax import lax
from jax.experimental import pallas as pl
from jax.experimental.pallas import tpu as pltpu
from jax.experimental.pallas import tpu_sc as plsc

N_NODES = 10000
NP = 10240
E = 640000
D = 128
NW = 32
RANGE = 320
CAP = 24576
EC = 2000
NCHUNK = E // EC
GB = 128
NEG_INF = float("-inf")


def _mesh():
    return plsc.VectorSubcoreMesh(core_axis_name="c", subcore_axis_name="s")


def _wid():
    return lax.axis_index("s") * 2 + lax.axis_index("c")


def _k2_body(a1_hbm, b1_hbm, dst_hbm, src_hbm,
             z_hbm, dstl_hbm, srcg_hbm, cnt_hbm,
             a1loc, sbuf, dbuf, dvm, svm, bbuf, cvm, sem):
    wid = _wid()
    lo = wid * RANGE
    pltpu.sync_copy(a1_hbm.at[pl.ds(lo, RANGE), :], a1loc)

    def step(j, cnt):
        d = dvm[pl.ds(j * 16, 16)]
        s = svm[pl.ds(j * 16, 16)]
        m = (d >= lo) & (d < lo + RANGE)
        pc = plsc.all_reduce_population_count(m)
        plsc.store_compressed(sbuf.at[pl.ds(cnt, 16)], s, mask=m)
        plsc.store_compressed(dbuf.at[pl.ds(cnt, 16)], d - lo, mask=m)
        return jnp.minimum(cnt + pc[0], CAP)

    def scan_chunk(c, cnt):
        pltpu.sync_copy(dst_hbm.at[pl.ds(c * EC, EC)], dvm)
        pltpu.sync_copy(src_hbm.at[pl.ds(c * EC, EC)], svm)
        return lax.fori_loop(0, EC // 16, step, cnt)

    cnt = lax.fori_loop(0, NCHUNK, scan_chunk, jnp.int32(0))

    ng = (cnt + GB - 1) >> 7

    s0 = jnp.full((16,), sbuf[pl.ds(0, 16)][0], jnp.int32)
    d0 = jnp.full((16,), dbuf[pl.ds(0, 16)][0], jnp.int32)
    for t in range(GB // 16):
        sbuf[pl.ds(cnt + t * 16, 16)] = s0
        dbuf[pl.ds(cnt + t * 16, 16)] = d0

    cvm[...] = jnp.full((16,), cnt, jnp.int32)
    pltpu.sync_copy(cvm, cnt_hbm.at[wid])
    pltpu.sync_copy(sbuf.at[pl.ds(0, CAP)], srcg_hbm.at[wid])
    pltpu.sync_copy(dbuf.at[pl.ds(0, CAP)], dstl_hbm.at[wid])

    def group(g, _):
        idx = sbuf.at[pl.ds(g * GB, GB)]
        pltpu.async_copy(b1_hbm.at[idx], bbuf, sem).wait()

        def sub(t, _):
            dlv = dbuf[pl.ds(g * GB + t * 16, 16)]
            for i in range(16):
                dl = dlv[i]
                e = t * 16 + i
                for k in range(8):
                    sl = pl.ds(k * 16, 16)
                    bbuf[e, sl] = jnp.maximum(bbuf[e, sl] + a1loc[dl, sl], 0.0)
            return 0

        lax.fori_loop(0, GB // 16, sub, 0)
        pltpu.sync_copy(bbuf, z_hbm.at[pl.ds(wid * CAP + g * GB, GB), :])
        return 0

    lax.fori_loop(0, ng, group, 0)


def _run_k2(a1, b1, dst, src):
    k = pl.kernel(
        _k2_body,
        out_type=(
            jax.ShapeDtypeStruct((NW * CAP, D), jnp.float32),
            jax.ShapeDtypeStruct((NW, CAP), jnp.int32),
            jax.ShapeDtypeStruct((NW, CAP), jnp.int32),
            jax.ShapeDtypeStruct((NW, 16), jnp.int32),
        ),
        mesh=_mesh(),
        compiler_params=pltpu.CompilerParams(needs_layout_passes=False),
        scratch_types=[
            pltpu.VMEM((RANGE, D), jnp.float32),
            pltpu.VMEM((CAP + GB,), jnp.int32),
            pltpu.VMEM((CAP + GB,), jnp.int32),
            pltpu.VMEM((EC,), jnp.int32),
            pltpu.VMEM((EC,), jnp.int32),
            pltpu.VMEM((GB, D), jnp.float32),
            pltpu.VMEM((16,), jnp.int32),
            pltpu.SemaphoreType.DMA,
        ],
    )
    return k(a1, b1, dst, src)


def _k4_body(h_hbm, dstl_hbm, cnt_hbm, acc_hbm, accv, hbuf, dgv, cvm, sem):
    wid = _wid()
    pltpu.sync_copy(cnt_hbm.at[wid], cvm)
    cnt = cvm[pl.ds(0, 16)][0]
    ng = (cnt + GB - 1) >> 7

    def initr(r, _):
        for k in range(8):
            accv[r, pl.ds(k * 16, 16)] = jnp.full((16,), NEG_INF, jnp.float32)
        return 0

    lax.fori_loop(0, RANGE, initr, 0)

    def group(g, _):
        pltpu.sync_copy(h_hbm.at[pl.ds(wid * CAP + g * GB, GB), :], hbuf)
        pltpu.sync_copy(dstl_hbm.at[wid, pl.ds(g * GB, GB)], dgv)

        def sub(t, _):
            dlv = dgv[pl.ds(t * 16, 16)]
            for i in range(16):
                dl = dlv[i]
                e = t * 16 + i
                for k in range(8):
                    sl = pl.ds(k * 16, 16)
                    accv[dl, sl] = jnp.maximum(accv[dl, sl], hbuf[e, sl])
            return 0

        lax.fori_loop(0, GB // 16, sub, 0)
        return 0

    lax.fori_loop(0, ng, group, 0)
    pltpu.sync_copy(accv, acc_hbm.at[pl.ds(wid * RANGE, RANGE), :])


def _run_k4(h, dstl, cnts):
    k = pl.kernel(
        _k4_body,
        out_type=jax.ShapeDtypeStruct((NP, D), jnp.float32),
        mesh=_mesh(),
        compiler_params=pltpu.CompilerParams(needs_layout_passes=False),
        scratch_types=[
            pltpu.VMEM((RANGE, D), jnp.float32),
            pltpu.VMEM((GB, D), jnp.float32),
            pltpu.VMEM((GB,), jnp.int32),
            pltpu.VMEM((16,), jnp.int32),
            pltpu.SemaphoreType.DMA,
        ],
    )
    return k(h, dstl, cnts)


GB6 = 64


def _k6_body(b2_hbm, dstl_hbm, srcg_hbm, cnt_hbm, m2_hbm,
             accv, bbuf, dgv, sgv, cvm, sem):
    wid = _wid()
    pltpu.sync_copy(cnt_hbm.at[wid], cvm)
    cnt = cvm[pl.ds(0, 16)][0]
    ng = (cnt + GB6 - 1) >> 6

    def initr(r, _):
        for k in range(16):
            accv[r, pl.ds(k * 16, 16)] = jnp.full((16,), NEG_INF, jnp.float32)
        return 0

    lax.fori_loop(0, RANGE, initr, 0)

    def group(g, _):
        pltpu.sync_copy(srcg_hbm.at[wid, pl.ds(g * GB6, GB6)], sgv)
        pltpu.async_copy(b2_hbm.at[sgv], bbuf, sem).wait()
        pltpu.sync_copy(dstl_hbm.at[wid, pl.ds(g * GB6, GB6)], dgv)

        def sub(t, _):
            dlv = dgv[pl.ds(t * 16, 16)]
            for i in range(16):
                dl = dlv[i]
                e = t * 16 + i
                for k in range(16):
                    sl = pl.ds(k * 16, 16)
                    accv[dl, sl] = jnp.maximum(accv[dl, sl], bbuf[e, sl])
            return 0

        lax.fori_loop(0, GB6 // 16, sub, 0)
        return 0

    lax.fori_loop(0, ng, group, 0)
    pltpu.sync_copy(accv, m2_hbm.at[pl.ds(wid * RANGE, RANGE), :])


def _run_k6(b2, dstl, srcg, cnts):
    k = pl.kernel(
        _k6_body,
        out_type=jax.ShapeDtypeStruct((NP, 2 * D), jnp.float32),
        mesh=_mesh(),
        compiler_params=pltpu.CompilerParams(needs_layout_passes=False),
        scratch_types=[
            pltpu.VMEM((RANGE, 2 * D), jnp.float32),
            pltpu.VMEM((GB6, 2 * D), jnp.float32),
            pltpu.VMEM((GB6,), jnp.int32),
            pltpu.VMEM((GB6,), jnp.int32),
            pltpu.VMEM((16,), jnp.int32),
            pltpu.SemaphoreType.DMA,
        ],
    )
    return k(b2, dstl, srcg, cnts)


BN = 512


def _k1_body(x_ref, w_ref, b_ref, oa_ref, ob_ref):
    acc = jnp.dot(x_ref[...], w_ref[...],
                  preferred_element_type=jnp.float32) + b_ref[...]
    oa_ref[...] = acc[:, :D]
    ob_ref[...] = acc[:, D:]


def _run_k1(xp, wcat, bcat):
    return pl.pallas_call(
        _k1_body,
        grid=(NP // BN,),
        in_specs=[
            pl.BlockSpec((BN, D), lambda i: (i, 0)),
            pl.BlockSpec((D, 2 * D), lambda i: (0, 0)),
            pl.BlockSpec((1, 2 * D), lambda i: (0, 0)),
        ],
        out_specs=[
            pl.BlockSpec((BN, D), lambda i: (i, 0)),
            pl.BlockSpec((BN, D), lambda i: (i, 0)),
        ],
        out_shape=[
            jax.ShapeDtypeStruct((NP, D), jnp.float32),
            jax.ShapeDtypeStruct((NP, D), jnp.float32),
        ],
    )(xp, wcat, bcat)


def _k3_body(z_ref, w_ref, h_ref):
    h_ref[...] = jnp.dot(jnp.maximum(z_ref[...], 0.0), w_ref[...],
                         preferred_element_type=jnp.float32)


def _run_k3(z, w1b):
    return pl.pallas_call(
        _k3_body,
        grid=(NW * CAP // BN,),
        in_specs=[
            pl.BlockSpec((BN, D), lambda i: (i, 0)),
            pl.BlockSpec((D, D), lambda i: (0, 0)),
        ],
        out_specs=pl.BlockSpec((BN, D), lambda i: (i, 0)),
        out_shape=jax.ShapeDtypeStruct((NW * CAP, D), jnp.float32),
    )(z, w1b)


def _k5_body(acc_ref, b1b_ref, w_ref, b_ref, oa_ref, ob_ref):
    a = acc_ref[...]
    x1 = jnp.where(a == NEG_INF, 0.0, a + b1b_ref[...])
    acc = jnp.dot(x1, w_ref[...], preferred_element_type=jnp.float32) + b_ref[...]
    oa_ref[...] = acc[:, :2 * D]
    ob_ref[...] = acc[:, 2 * D:]


def _run_k5(acc1, b1b, wcat, bcat):
    return pl.pallas_call(
        _k5_body,
        grid=(NP // BN,),
        in_specs=[
            pl.BlockSpec((BN, D), lambda i: (i, 0)),
            pl.BlockSpec((1, D), lambda i: (0, 0)),
            pl.BlockSpec((D, 4 * D), lambda i: (0, 0)),
            pl.BlockSpec((1, 4 * D), lambda i: (0, 0)),
        ],
        out_specs=[
            pl.BlockSpec((BN, 2 * D), lambda i: (i, 0)),
            pl.BlockSpec((BN, 2 * D), lambda i: (i, 0)),
        ],
        out_shape=[
            jax.ShapeDtypeStruct((NP, 2 * D), jnp.float32),
            jax.ShapeDtypeStruct((NP, 2 * D), jnp.float32),
        ],
    )(acc1, b1b, wcat, bcat)


def _k7_body(a2_ref, m2_ref, w3_ref, b3_ref, batch_ref, g_ref):
    pid = pl.program_id(0)

    @pl.when(pid == 0)
    def _():
        g_ref[...] = jnp.full((16, 4 * D), NEG_INF, jnp.float32)

    m2 = m2_ref[...]
    x2 = jnp.where(m2 == NEG_INF, 0.0, a2_ref[...] + m2)
    x3 = jnp.dot(x2, w3_ref[...], preferred_element_type=jnp.float32) + b3_ref[...]
    b = batch_ref[0, 0, :]
    rid = pid * BN + lax.broadcasted_iota(jnp.int32, (BN, 1), 0)
    valid = rid < N_NODES
    rows = []
    for gid in range(16):
        mrow = (b[:, None] == gid) & valid
        contrib = jnp.where(mrow, x3, NEG_INF)
        rows.append(jnp.max(contrib, axis=0, keepdims=True))
    g_ref[...] = jnp.maximum(g_ref[...], jnp.concatenate(rows, axis=0))


def _run_k7(a2, m2, w3, b3, batch3):
    return pl.pallas_call(
        _k7_body,
        grid=(NP // BN,),
        in_specs=[
            pl.BlockSpec((BN, 2 * D), lambda i: (i, 0)),
            pl.BlockSpec((BN, 2 * D), lambda i: (i, 0)),
            pl.BlockSpec((2 * D, 4 * D), lambda i: (0, 0)),
            pl.BlockSpec((1, 4 * D), lambda i: (0, 0)),
            pl.BlockSpec((1, 1, BN), lambda i: (i, 0, 0)),
        ],
        out_specs=pl.BlockSpec((16, 4 * D), lambda i: (0, 0)),
        out_shape=jax.ShapeDtypeStruct((16, 4 * D), jnp.float32),
    )(a2, m2, w3, b3, batch3)


def _k8_body(g_ref, wf1_ref, bf1_ref, wf2_ref, bf2_ref, o_ref):
    g = g_ref[...]
    gs = jnp.where(g == NEG_INF, 0.0, g)
    h = jnp.maximum(jnp.dot(gs, wf1_ref[...],
                            preferred_element_type=jnp.float32) + bf1_ref[...], 0.0)
    z = jnp.dot(h, wf2_ref[...], preferred_element_type=jnp.float32) + bf2_ref[...]
    colid = lax.broadcasted_iota(jnp.int32, (16, D), 1)
    zm = jnp.where(colid < 40, z, NEG_INF)
    mx = jnp.max(zm, axis=1, keepdims=True)
    lse = jnp.log(jnp.sum(jnp.exp(zm - mx), axis=1, keepdims=True)) + mx
    o_ref[...] = zm - lse


def _run_k8(g, wf1, bf1, wf2p, bf2p):
    return pl.pallas_call(
        _k8_body,
        grid=(1,),
        in_specs=[
            pl.BlockSpec((16, 4 * D), lambda i: (0, 0)),
            pl.BlockSpec((4 * D, 64), lambda i: (0, 0)),
            pl.BlockSpec((1, 64), lambda i: (0, 0)),
            pl.BlockSpec((64, D), lambda i: (0, 0)),
            pl.BlockSpec((1, D), lambda i: (0, 0)),
        ],
        out_specs=pl.BlockSpec((16, D), lambda i: (0, 0)),
        out_shape=jax.ShapeDtypeStruct((16, D), jnp.float32),
    )(g, wf1, bf1, wf2p, bf2p)


def kernel(x, edge_index, batch, W1a, b1a, W1b, b1b, W2, b2, W3, b3,
           Wf1, bf1, Wf2, bf2):
    src = jnp.concatenate([edge_index[0], edge_index[1]])
    dst = jnp.concatenate([edge_index[1], edge_index[0]])
    xp = jnp.pad(x, ((0, NP - N_NODES), (0, 0)))
    batch3 = jnp.pad(batch, (0, NP - N_NODES)).reshape(NP // BN, 1, BN)

    wcat1 = jnp.concatenate([W1a[:D] - W1a[D:], W1a[D:]], axis=1)
    bcat1 = jnp.concatenate([jnp.zeros((D,), jnp.float32), b1a]).reshape(1, 2 * D)
    wcat2 = jnp.concatenate([W2[:D] - W2[D:], W2[D:]], axis=1)
    bcat2 = jnp.concatenate([b2, jnp.zeros((2 * D,), jnp.float32)]).reshape(1, 4 * D)
    wf2p = jnp.pad(Wf2, ((0, 0), (0, D - 40)))
    bf2p = jnp.pad(bf2, (0, D - 40)).reshape(1, D)

    a1, b1 = _run_k1(xp, wcat1, bcat1)
    z, dstl, srcg, cnts = _run_k2(a1, b1, dst, src)
    h = _run_k3(z, W1b)
    acc1 = _run_k4(h, dstl, cnts)
    a2, b2t = _run_k5(acc1, b1b.reshape(1, D), wcat2, bcat2)
    m2 = _run_k6(b2t, dstl, srcg, cnts)
    g = _run_k7(a2, m2, W3, b3.reshape(1, 4 * D), batch3)
    out = _run_k8(g, Wf1, bf1.reshape(1, 64), wf2p, bf2p)
    return out[:, :40]


# speedup vs baseline: 3.6751x; 1.3876x over previous
import jax
import jax.numpy as jnp
from jax import lax
from jax.experimental import pallas as pl
from jax.experimental.pallas import tpu as pltpu
from jax.experimental.pallas import tpu_sc as plsc

N_NODES = 10000
NP = 10240
E = 640000
D = 128
NW = 32
RANGE = 320
CAP = 24576
EC = 4000
NCHUNK = E // EC
GB = 128
GB6 = 64
LC = 2048
NEG_INF = float("-inf")


def _mesh():
    return plsc.VectorSubcoreMesh(core_axis_name="c", subcore_axis_name="s")


def _wid():
    return lax.axis_index("s") * 2 + lax.axis_index("c")


_SC_PARAMS = pltpu.CompilerParams(needs_layout_passes=False)


def _k0_body(dst_hbm, src_hbm, dstl_hbm, srcg_hbm, cnt_hbm,
             sbuf, dbuf, dvm0, svm0, dvm1, svm1, cvm, sem0, sem1):
    wid = _wid()
    lo = wid * RANGE

    def issue(c, dvm, svm, sem):
        pltpu.async_copy(dst_hbm.at[pl.ds(c * EC, EC)], dvm, sem)
        pltpu.async_copy(src_hbm.at[pl.ds(c * EC, EC)], svm, sem)

    def wait(dvm, svm, sem):
        pltpu.make_async_copy(dst_hbm.at[pl.ds(0, EC)], dvm, sem).wait()
        pltpu.make_async_copy(src_hbm.at[pl.ds(0, EC)], svm, sem).wait()

    issue(0, dvm0, svm0, sem0)
    issue(1, dvm1, svm1, sem1)

    def scan(dvm, svm, cnt0):
        @plsc.parallel_loop(0, EC // 16, unroll=10, carry=cnt0)
        def body(j, cnt):
            d = dvm[pl.ds(j * 16, 16)]
            s = svm[pl.ds(j * 16, 16)]
            m = (d >= lo) & (d < lo + RANGE)
            pc = plsc.all_reduce_population_count(m)
            plsc.store_compressed(sbuf.at[pl.ds(cnt, 16)], s, mask=m)
            plsc.store_compressed(dbuf.at[pl.ds(cnt, 16)], d - lo, mask=m)
            return jnp.minimum(cnt + pc[0], CAP)

        return body

    def chunk_pair(c2, cnt):
        c = c2 * 2
        wait(dvm0, svm0, sem0)
        cnt = scan(dvm0, svm0, cnt)

        @pl.when(c + 2 < NCHUNK)
        def _():
            issue(c + 2, dvm0, svm0, sem0)

        wait(dvm1, svm1, sem1)
        cnt = scan(dvm1, svm1, cnt)

        @pl.when(c + 3 < NCHUNK)
        def _():
            issue(c + 3, dvm1, svm1, sem1)

        return cnt

    cnt = lax.fori_loop(0, NCHUNK // 2, chunk_pair, jnp.int32(0))

    s0 = jnp.full((16,), sbuf[pl.ds(0, 16)][0], jnp.int32)
    d0 = jnp.full((16,), dbuf[pl.ds(0, 16)][0], jnp.int32)
    for t in range(GB // 16):
        sbuf[pl.ds(cnt + t * 16, 16)] = s0
        dbuf[pl.ds(cnt + t * 16, 16)] = d0

    cvm[...] = jnp.full((16,), cnt, jnp.int32)
    pltpu.sync_copy(cvm, cnt_hbm.at[wid])
    pltpu.sync_copy(sbuf.at[pl.ds(0, CAP)], srcg_hbm.at[wid])
    pltpu.sync_copy(dbuf.at[pl.ds(0, CAP)], dstl_hbm.at[wid])


def _run_k0(dst, src):
    k = pl.kernel(
        _k0_body,
        out_type=(
            jax.ShapeDtypeStruct((NW, CAP), jnp.int32),
            jax.ShapeDtypeStruct((NW, CAP), jnp.int32),
            jax.ShapeDtypeStruct((NW, 16), jnp.int32),
        ),
        mesh=_mesh(),
        compiler_params=_SC_PARAMS,
        scratch_types=[
            pltpu.VMEM((CAP + 256,), jnp.int32),
            pltpu.VMEM((CAP + 256,), jnp.int32),
            pltpu.VMEM((EC,), jnp.int32),
            pltpu.VMEM((EC,), jnp.int32),
            pltpu.VMEM((EC,), jnp.int32),
            pltpu.VMEM((EC,), jnp.int32),
            pltpu.VMEM((16,), jnp.int32),
            pltpu.SemaphoreType.DMA,
            pltpu.SemaphoreType.DMA,
        ],
    )
    return k(dst, src)


def _k2_body(a1_hbm, b1_hbm, dstl_hbm, srcg_hbm, cnt_hbm, z_hbm,
             a1loc, scb, dcb, bb0, bb1, zb0, zb1, cvm,
             gs0, gs1, ws0, ws1):
    wid = _wid()
    lo = wid * RANGE
    pltpu.sync_copy(a1_hbm.at[pl.ds(lo, RANGE), :], a1loc)
    pltpu.sync_copy(cnt_hbm.at[wid], cvm)
    cnt = cvm[pl.ds(0, 16)][0]
    ng = (cnt + GB - 1) >> 7
    ncl = (cnt + LC - 1) >> 11
    gpc = LC // GB

    bbs = (bb0, bb1)
    zbs = (zb0, zb1)
    gss = (gs0, gs1)
    wss = (ws0, ws1)

    def load_chunk(c):
        pltpu.sync_copy(srcg_hbm.at[wid, pl.ds(c * LC, LC)], scb.at[c % 2])
        pltpu.sync_copy(dstl_hbm.at[wid, pl.ds(c * LC, LC)], dcb.at[c % 2])

    def issue_gather(g, b):
        cb = (g // gpc) % 2
        idx = scb.at[cb, pl.ds((g % gpc) * GB, GB)]
        pltpu.async_copy(b1_hbm.at[idx], bbs[b], gss[b])

    load_chunk(0)

    @pl.when(ncl > 1)
    def _():
        load_chunk(1)

    @pl.when(ng > 0)
    def _():
        issue_gather(0, 0)

    def do_group(g, b):
        bbuf = bbs[b]
        zbuf = zbs[b]
        ob = 1 - b
        pltpu.make_async_copy(
            b1_hbm.at[scb.at[0, pl.ds(0, GB)]], bbuf, gss[b]).wait()

        @pl.when((g % gpc == 0) & ((g // gpc) + 1 < ncl))
        def _():
            load_chunk((g // gpc) + 1)

        @pl.when(g + 1 < ng)
        def _():
            @pl.when(g >= 1)
            def _():
                pltpu.make_async_copy(
                    zbs[ob], z_hbm.at[pl.ds(0, GB), :], wss[ob]).wait()

            issue_gather(g + 1, ob)

        cb = (g // gpc) % 2
        base = (g % gpc) * GB

        def sub(t, _):
            dlv = dcb[cb, pl.ds(base + t * 16, 16)]
            for i in range(16):
                dl = dlv[i]
                e = t * 16 + i
                for k in range(8):
                    sl = pl.ds(k * 16, 16)
                    zbuf[e, sl] = jnp.maximum(
                        bbuf[e, sl] + a1loc[dl, sl], 0.0)
            return 0

        lax.fori_loop(0, GB // 16, sub, 0)
        pltpu.async_copy(zbuf, z_hbm.at[pl.ds(wid * CAP + g * GB, GB), :],
                         wss[b])

    def pair(g2, _):
        g = g2 * 2
        do_group(g, 0)

        @pl.when(g + 1 < ng)
        def _():
            do_group(g + 1, 1)

        return 0

    lax.fori_loop(0, (ng + 1) >> 1, pair, 0)

    for s in (0, 1):
        @pl.when((ng >= 2) | ((ng >= 1) & ((ng - 1) % 2 == s)))
        def _(s=s):
            pltpu.make_async_copy(
                zbs[s], z_hbm.at[pl.ds(0, GB), :], wss[s]).wait()


def _run_k2(a1, b1, dstl, srcg, cnts):
    k = pl.kernel(
        _k2_body,
        out_type=jax.ShapeDtypeStruct((NW * CAP, D), jnp.float32),
        mesh=_mesh(),
        compiler_params=_SC_PARAMS,
        scratch_types=[
            pltpu.VMEM((RANGE, D), jnp.float32),
            pltpu.VMEM((2, LC), jnp.int32),
            pltpu.VMEM((2, LC), jnp.int32),
            pltpu.VMEM((GB, D), jnp.float32),
            pltpu.VMEM((GB, D), jnp.float32),
            pltpu.VMEM((GB, D), jnp.float32),
            pltpu.VMEM((GB, D), jnp.float32),
            pltpu.VMEM((16,), jnp.int32),
            pltpu.SemaphoreType.DMA,
            pltpu.SemaphoreType.DMA,
            pltpu.SemaphoreType.DMA,
            pltpu.SemaphoreType.DMA,
        ],
    )
    return k(a1, b1, dstl, srcg, cnts)


def _k4_body(h_hbm, dstl_hbm, cnt_hbm, acc_hbm,
             av0, av1, dcb, hb0, hb1, cvm, gs0, gs1):
    wid = _wid()
    pltpu.sync_copy(cnt_hbm.at[wid], cvm)
    cnt = cvm[pl.ds(0, 16)][0]
    ng = (cnt + GB - 1) >> 7
    ncl = (cnt + LC - 1) >> 11
    gpc = LC // GB

    avs = (av0, av1)
    hbs = (hb0, hb1)
    gss = (gs0, gs1)

    def initr(r, _):
        for q in range(2):
            for k2 in range(4):
                avs[q][r, pl.ds(k2 * 16, 16)] = jnp.full(
                    (16,), NEG_INF, jnp.float32)
        return 0

    lax.fori_loop(0, RANGE, initr, 0)

    def load_chunk(c):
        pltpu.sync_copy(dstl_hbm.at[wid, pl.ds(c * LC, LC)], dcb.at[c % 2])

    def issue_gather(g, b):
        pltpu.async_copy(
            h_hbm.at[pl.ds(wid * CAP + g * GB, GB), :], hbs[b], gss[b])

    load_chunk(0)

    @pl.when(ncl > 1)
    def _():
        load_chunk(1)

    @pl.when(ng > 0)
    def _():
        issue_gather(0, 0)

    @pl.when(ng > 1)
    def _():
        issue_gather(1, 1)

    def do_group(g, b):
        hbuf = hbs[b]
        pltpu.make_async_copy(
            h_hbm.at[pl.ds(0, GB), :], hbuf, gss[b]).wait()

        @pl.when((g % gpc == 0) & ((g // gpc) + 1 < ncl))
        def _():
            load_chunk((g // gpc) + 1)

        cb = (g // gpc) % 2
        base = (g % gpc) * GB

        def sub(t, _):
            dlv = dcb[cb, pl.ds(base + t * 16, 16)]
            for i in range(16):
                dl = dlv[i]
                e = t * 16 + i
                for q in range(2):
                    for k2 in range(4):
                        sa = pl.ds(k2 * 16, 16)
                        sh = pl.ds(q * 64 + k2 * 16, 16)
                        avs[q][dl, sa] = jnp.maximum(
                            avs[q][dl, sa], hbuf[e, sh])
            return 0

        lax.fori_loop(0, GB // 16, sub, 0)

        @pl.when(g + 2 < ng)
        def _():
            issue_gather(g + 2, b)

    def pair(g2, _):
        g = g2 * 2
        do_group(g, 0)

        @pl.when(g + 1 < ng)
        def _():
            do_group(g + 1, 1)

        return 0

    lax.fori_loop(0, (ng + 1) >> 1, pair, 0)

    for rb, rcount in ((0, 128), (1, 128), (2, 64)):
        def cprow(r, _, rb=rb):
            for q in range(2):
                for k2 in range(4):
                    hb0[r, pl.ds(q * 64 + k2 * 16, 16)] = (
                        avs[q][rb * 128 + r, pl.ds(k2 * 16, 16)])
            return 0

        lax.fori_loop(0, rcount, cprow, 0)
        pltpu.sync_copy(
            hb0.at[pl.ds(0, rcount), :],
            acc_hbm.at[pl.ds(wid * RANGE + rb * 128, rcount), :])


def _run_k4(h, dstl, cnts):
    k = pl.kernel(
        _k4_body,
        out_type=jax.ShapeDtypeStruct((NP, D), jnp.float32),
        mesh=_mesh(),
        compiler_params=_SC_PARAMS,
        scratch_types=[
            pltpu.VMEM((RANGE, 64), jnp.float32),
            pltpu.VMEM((RANGE, 64), jnp.float32),
            pltpu.VMEM((2, LC), jnp.int32),
            pltpu.VMEM((GB, D), jnp.float32),
            pltpu.VMEM((GB, D), jnp.float32),
            pltpu.VMEM((16,), jnp.int32),
            pltpu.SemaphoreType.DMA,
            pltpu.SemaphoreType.DMA,
        ],
    )
    return k(h, dstl, cnts)


def _k6_body(b2_hbm, dstl_hbm, srcg_hbm, cnt_hbm, m2_hbm,
             av0, av1, scb, dcb, bb0, bb1, cvm, gs0, gs1):
    wid = _wid()
    pltpu.sync_copy(cnt_hbm.at[wid], cvm)
    cnt = cvm[pl.ds(0, 16)][0]
    ng = (cnt + GB6 - 1) >> 6
    ncl = (cnt + LC - 1) >> 11
    gpc = LC // GB6

    avs = (av0, av1)
    bbs = (bb0, bb1)
    gss = (gs0, gs1)

    def initr(r, _):
        for q in range(2):
            for k2 in range(8):
                avs[q][r, pl.ds(k2 * 16, 16)] = jnp.full(
                    (16,), NEG_INF, jnp.float32)
        return 0

    lax.fori_loop(0, RANGE, initr, 0)

    def load_chunk(c):
        pltpu.sync_copy(srcg_hbm.at[wid, pl.ds(c * LC, LC)], scb.at[c % 2])
        pltpu.sync_copy(dstl_hbm.at[wid, pl.ds(c * LC, LC)], dcb.at[c % 2])

    def issue_gather(g, b):
        cb = (g // gpc) % 2
        idx = scb.at[cb, pl.ds((g % gpc) * GB6, GB6)]
        pltpu.async_copy(b2_hbm.at[idx], bbs[b], gss[b])

    load_chunk(0)

    @pl.when(ncl > 1)
    def _():
        load_chunk(1)

    @pl.when(ng > 0)
    def _():
        issue_gather(0, 0)

    @pl.when(ng > 1)
    def _():
        issue_gather(1, 1)

    def do_group(g, b):
        bbuf = bbs[b]
        pltpu.make_async_copy(
            b2_hbm.at[scb.at[0, pl.ds(0, GB6)]], bbuf, gss[b]).wait()

        @pl.when((g % gpc == 0) & ((g // gpc) + 1 < ncl))
        def _():
            load_chunk((g // gpc) + 1)

        cb = (g // gpc) % 2
        base = (g % gpc) * GB6

        def sub(t, _):
            dlv = dcb[cb, pl.ds(base + t * 16, 16)]
            for i in range(16):
                dl = dlv[i]
                e = t * 16 + i
                for q in range(2):
                    for k2 in range(8):
                        sa = pl.ds(k2 * 16, 16)
                        sh = pl.ds(q * 128 + k2 * 16, 16)
                        avs[q][dl, sa] = jnp.maximum(
                            avs[q][dl, sa], bbuf[e, sh])
            return 0

        lax.fori_loop(0, GB6 // 16, sub, 0)

        @pl.when(g + 2 < ng)
        def _():
            issue_gather(g + 2, b)

    def pair(g2, _):
        g = g2 * 2
        do_group(g, 0)

        @pl.when(g + 1 < ng)
        def _():
            do_group(g + 1, 1)

        return 0

    lax.fori_loop(0, (ng + 1) >> 1, pair, 0)

    for rb in range(5):
        def cprow(r, _, rb=rb):
            for q in range(2):
                for k2 in range(8):
                    bb0[r, pl.ds(q * 128 + k2 * 16, 16)] = (
                        avs[q][rb * 64 + r, pl.ds(k2 * 16, 16)])
            return 0

        lax.fori_loop(0, GB6, cprow, 0)
        pltpu.sync_copy(
            bb0,
            m2_hbm.at[pl.ds(wid * RANGE + rb * 64, 64), :])


def _run_k6(b2, dstl, srcg, cnts):
    k = pl.kernel(
        _k6_body,
        out_type=jax.ShapeDtypeStruct((NP, 2 * D), jnp.float32),
        mesh=_mesh(),
        compiler_params=_SC_PARAMS,
        scratch_types=[
            pltpu.VMEM((RANGE, D), jnp.float32),
            pltpu.VMEM((RANGE, D), jnp.float32),
            pltpu.VMEM((2, LC), jnp.int32),
            pltpu.VMEM((2, LC), jnp.int32),
            pltpu.VMEM((GB6, 2 * D), jnp.float32),
            pltpu.VMEM((GB6, 2 * D), jnp.float32),
            pltpu.VMEM((16,), jnp.int32),
            pltpu.SemaphoreType.DMA,
            pltpu.SemaphoreType.DMA,
        ],
    )
    return k(b2, dstl, srcg, cnts)


BN = 512


def _k1_body(x_ref, w_ref, b_ref, oa_ref, ob_ref):
    acc = jnp.dot(x_ref[...], w_ref[...],
                  preferred_element_type=jnp.float32) + b_ref[...]
    oa_ref[...] = acc[:, :D]
    ob_ref[...] = acc[:, D:]


def _run_k1(xp, wcat, bcat):
    return pl.pallas_call(
        _k1_body,
        grid=(NP // BN,),
        in_specs=[
            pl.BlockSpec((BN, D), lambda i: (i, 0)),
            pl.BlockSpec((D, 2 * D), lambda i: (0, 0)),
            pl.BlockSpec((1, 2 * D), lambda i: (0, 0)),
        ],
        out_specs=[
            pl.BlockSpec((BN, D), lambda i: (i, 0)),
            pl.BlockSpec((BN, D), lambda i: (i, 0)),
        ],
        out_shape=[
            jax.ShapeDtypeStruct((NP, D), jnp.float32),
            jax.ShapeDtypeStruct((NP, D), jnp.float32),
        ],
    )(xp, wcat, bcat)


def _k3_body(z_ref, w_ref, h_ref):
    h_ref[...] = jnp.dot(jnp.maximum(z_ref[...], 0.0), w_ref[...],
                         preferred_element_type=jnp.float32)


def _run_k3(z, w1b):
    return pl.pallas_call(
        _k3_body,
        grid=(NW * CAP // BN,),
        in_specs=[
            pl.BlockSpec((BN, D), lambda i: (i, 0)),
            pl.BlockSpec((D, D), lambda i: (0, 0)),
        ],
        out_specs=pl.BlockSpec((BN, D), lambda i: (i, 0)),
        out_shape=jax.ShapeDtypeStruct((NW * CAP, D), jnp.float32),
    )(z, w1b)


def _k5_body(acc_ref, b1b_ref, w_ref, b_ref, oa_ref, ob_ref):
    a = acc_ref[...]
    x1 = jnp.where(a == NEG_INF, 0.0, a + b1b_ref[...])
    acc = jnp.dot(x1, w_ref[...], preferred_element_type=jnp.float32) + b_ref[...]
    oa_ref[...] = acc[:, :2 * D]
    ob_ref[...] = acc[:, 2 * D:]


def _run_k5(acc1, b1b, wcat, bcat):
    return pl.pallas_call(
        _k5_body,
        grid=(NP // BN,),
        in_specs=[
            pl.BlockSpec((BN, D), lambda i: (i, 0)),
            pl.BlockSpec((1, D), lambda i: (0, 0)),
            pl.BlockSpec((D, 4 * D), lambda i: (0, 0)),
            pl.BlockSpec((1, 4 * D), lambda i: (0, 0)),
        ],
        out_specs=[
            pl.BlockSpec((BN, 2 * D), lambda i: (i, 0)),
            pl.BlockSpec((BN, 2 * D), lambda i: (i, 0)),
        ],
        out_shape=[
            jax.ShapeDtypeStruct((NP, 2 * D), jnp.float32),
            jax.ShapeDtypeStruct((NP, 2 * D), jnp.float32),
        ],
    )(acc1, b1b, wcat, bcat)


def _k7_body(a2_ref, m2_ref, w3_ref, b3_ref, batch_ref, g_ref):
    pid = pl.program_id(0)

    @pl.when(pid == 0)
    def _():
        g_ref[...] = jnp.full((16, 4 * D), NEG_INF, jnp.float32)

    m2 = m2_ref[...]
    x2 = jnp.where(m2 == NEG_INF, 0.0, a2_ref[...] + m2)
    x3 = jnp.dot(x2, w3_ref[...], preferred_element_type=jnp.float32) + b3_ref[...]
    b = batch_ref[0, 0, :]
    rid = pid * BN + lax.broadcasted_iota(jnp.int32, (BN, 1), 0)
    valid = rid < N_NODES
    rows = []
    for gid in range(16):
        mrow = (b[:, None] == gid) & valid
        contrib = jnp.where(mrow, x3, NEG_INF)
        rows.append(jnp.max(contrib, axis=0, keepdims=True))
    g_ref[...] = jnp.maximum(g_ref[...], jnp.concatenate(rows, axis=0))


def _run_k7(a2, m2, w3, b3, batch3):
    return pl.pallas_call(
        _k7_body,
        grid=(NP // BN,),
        in_specs=[
            pl.BlockSpec((BN, 2 * D), lambda i: (i, 0)),
            pl.BlockSpec((BN, 2 * D), lambda i: (i, 0)),
            pl.BlockSpec((2 * D, 4 * D), lambda i: (0, 0)),
            pl.BlockSpec((1, 4 * D), lambda i: (0, 0)),
            pl.BlockSpec((1, 1, BN), lambda i: (i, 0, 0)),
        ],
        out_specs=pl.BlockSpec((16, 4 * D), lambda i: (0, 0)),
        out_shape=jax.ShapeDtypeStruct((16, 4 * D), jnp.float32),
    )(a2, m2, w3, b3, batch3)


def _k8_body(g_ref, wf1_ref, bf1_ref, wf2_ref, bf2_ref, o_ref):
    g = g_ref[...]
    gs = jnp.where(g == NEG_INF, 0.0, g)
    h = jnp.maximum(jnp.dot(gs, wf1_ref[...],
                            preferred_element_type=jnp.float32) + bf1_ref[...], 0.0)
    z = jnp.dot(h, wf2_ref[...], preferred_element_type=jnp.float32) + bf2_ref[...]
    colid = lax.broadcasted_iota(jnp.int32, (16, D), 1)
    zm = jnp.where(colid < 40, z, NEG_INF)
    mx = jnp.max(zm, axis=1, keepdims=True)
    lse = jnp.log(jnp.sum(jnp.exp(zm - mx), axis=1, keepdims=True)) + mx
    o_ref[...] = zm - lse


def _run_k8(g, wf1, bf1, wf2p, bf2p):
    return pl.pallas_call(
        _k8_body,
        grid=(1,),
        in_specs=[
            pl.BlockSpec((16, 4 * D), lambda i: (0, 0)),
            pl.BlockSpec((4 * D, 64), lambda i: (0, 0)),
            pl.BlockSpec((1, 64), lambda i: (0, 0)),
            pl.BlockSpec((64, D), lambda i: (0, 0)),
            pl.BlockSpec((1, D), lambda i: (0, 0)),
        ],
        out_specs=pl.BlockSpec((16, D), lambda i: (0, 0)),
        out_shape=jax.ShapeDtypeStruct((16, D), jnp.float32),
    )(g, wf1, bf1, wf2p, bf2p)


def kernel(x, edge_index, batch, W1a, b1a, W1b, b1b, W2, b2, W3, b3,
           Wf1, bf1, Wf2, bf2):
    src = jnp.concatenate([edge_index[0], edge_index[1]])
    dst = jnp.concatenate([edge_index[1], edge_index[0]])
    xp = jnp.pad(x, ((0, NP - N_NODES), (0, 0)))
    batch3 = jnp.pad(batch, (0, NP - N_NODES)).reshape(NP // BN, 1, BN)

    wcat1 = jnp.concatenate([W1a[:D] - W1a[D:], W1a[D:]], axis=1)
    bcat1 = jnp.concatenate([jnp.zeros((D,), jnp.float32), b1a]).reshape(1, 2 * D)
    wcat2 = jnp.concatenate([W2[:D] - W2[D:], W2[D:]], axis=1)
    bcat2 = jnp.concatenate([b2, jnp.zeros((2 * D,), jnp.float32)]).reshape(1, 4 * D)
    wf2p = jnp.pad(Wf2, ((0, 0), (0, D - 40)))
    bf2p = jnp.pad(bf2, (0, D - 40)).reshape(1, D)

    a1, b1 = _run_k1(xp, wcat1, bcat1)
    dstl, srcg, cnts = _run_k0(dst, src)
    z = _run_k2(a1, b1, dstl, srcg, cnts)
    h = _run_k3(z, W1b)
    acc1 = _run_k4(h, dstl, cnts)
    a2, b2t = _run_k5(acc1, b1b.reshape(1, D), wcat2, bcat2)
    m2 = _run_k6(b2t, dstl, srcg, cnts)
    g = _run_k7(a2, m2, W3, b3.reshape(1, 4 * D), batch3)
    out = _run_k8(g, Wf1, bf1.reshape(1, 64), wf2p, bf2p)
    return out[:, :40]
